# baseline (speedup 1.0000x reference)
import functools

import jax
import jax.numpy as jnp
from jax import lax
from jax.experimental import pallas as pl
from jax.experimental.pallas import tpu as pltpu
from jax.experimental.pallas import tpu_sc as plsc

K = 1024
DIM = 2048
KNN = 8
Q = 32
KT = 256
NSTEPS = K // KT
L = 16
NVREG = K // L


def _tc_body(query_ref, anchor_ref, out_ref, logq_ref):
    i = pl.program_id(0)

    @pl.when(i == 0)
    def _():
        logq_ref[...] = jnp.log(query_ref[...])

    a = anchor_ref[...]
    log_a = jnp.log(a)
    self_term = jnp.sum(a * log_a, axis=1, keepdims=True)
    cross = lax.dot_general(
        a, logq_ref[...], (((1,), (1,)), ((), ())),
        preferred_element_type=jnp.float32)
    out_ref[...] = lax.transpose((self_term - cross) / DIM, (1, 0))


def _merge(a, b):
    ka, va = a
    kb, vb = b
    rkb = lax.rev(kb, (0,))
    rvb = lax.rev(vb, (0,))
    pred = ka <= rkb
    ck = jnp.where(pred, ka, rkb)
    cv = jnp.where(pred, va, rvb)
    return plsc.sort_key_val(ck, cv)


def _sc_body(dist_hbm, label_hbm, out_hbm, dist_v, lab_v, vote_v):
    wid = lax.axis_index("s") * 2 + lax.axis_index("c")
    pltpu.sync_copy(dist_hbm.at[wid], dist_v)
    pltpu.sync_copy(label_hbm, lab_v)
    lane = lax.broadcasted_iota(jnp.int32, (L,), 0)
    lists = []
    for j in range(NVREG):
        kj = dist_v[pl.ds(j * L, L)]
        lists.append(plsc.sort_key_val(kj, lane + j * L))
    while len(lists) > 1:
        lists = [_merge(lists[i], lists[i + 1]) for i in range(0, len(lists), 2)]
    _, top_idx = lists[0]
    labs = plsc.load_gather(lab_v, [top_idx])
    s = jnp.sum(jnp.where(lane < KNN, labs, 0))
    vote_v[...] = jnp.broadcast_to(jnp.where(s >= KNN // 2 + 1, 1, 0), (L,))
    pltpu.sync_copy(vote_v, out_hbm.at[wid])


@functools.partial(
    pl.kernel,
    out_type=jax.ShapeDtypeStruct((Q, L), jnp.int32),
    mesh=plsc.VectorSubcoreMesh(core_axis_name="c", subcore_axis_name="s"),
    compiler_params=pltpu.CompilerParams(needs_layout_passes=False),
    scratch_types=[
        pltpu.VMEM((K,), jnp.float32),
        pltpu.VMEM((K,), jnp.int32),
        pltpu.VMEM((L,), jnp.int32),
    ],
)
def _sc_topk_vote(dist_hbm, label_hbm, out_hbm, dist_v, lab_v, vote_v):
    _sc_body(dist_hbm, label_hbm, out_hbm, dist_v, lab_v, vote_v)


@jax.jit
def kernel(query, queue_anchor, queue_label):
    dist = pl.pallas_call(
        _tc_body,
        grid=(NSTEPS,),
        in_specs=[
            pl.BlockSpec((Q, DIM), lambda i: (0, 0)),
            pl.BlockSpec((KT, DIM), lambda i: (i, 0)),
        ],
        out_specs=pl.BlockSpec((Q, KT), lambda i: (0, i)),
        out_shape=jax.ShapeDtypeStruct((Q, K), jnp.float32),
        scratch_shapes=[
            pltpu.VMEM((Q, DIM), jnp.float32),
        ],
    )(query, queue_anchor)
    votes = _sc_topk_vote(dist, queue_label.astype(jnp.int32))
    return votes[:, 0]


# speedup vs baseline: 2.5053x; 2.5053x over previous
import functools

import jax
import jax.numpy as jnp
from jax import lax
from jax.experimental import pallas as pl
from jax.experimental.pallas import tpu as pltpu

K = 1024
DIM = 2048
KNN = 8
Q = 32
KT = 256
NSTEPS = K // KT


def _tc_body(query_ref, anchor_ref, label_ref, out_ref, dist_ref, logq_ref):
    i = pl.program_id(0)

    @pl.when(i == 0)
    def _():
        logq_ref[...] = jnp.log(query_ref[...])

    a = anchor_ref[...]
    log_a = jnp.log(a)
    self_term = jnp.sum(a * log_a, axis=1, keepdims=True)
    cross = lax.dot_general(
        a, logq_ref[...], (((1,), (1,)), ((), ())),
        preferred_element_type=jnp.float32)
    dist_ref[:, pl.ds(i * KT, KT)] = lax.transpose(
        (self_term - cross) / DIM, (1, 0))

    @pl.when(i == NSTEPS - 1)
    def _():
        d = dist_ref[...]
        iota = lax.broadcasted_iota(jnp.int32, (Q, K), 1)
        labels = label_ref[...]
        s = jnp.zeros((Q, 1), jnp.float32)
        for _ in range(KNN):
            m = jnp.min(d, axis=1, keepdims=True)
            idx = jnp.min(jnp.where(d == m, iota, K), axis=1, keepdims=True)
            sel = iota == idx
            s = s + jnp.sum(jnp.where(sel, labels, 0.0), axis=1, keepdims=True)
            d = jnp.where(sel, jnp.inf, d)
        out_ref[...] = (s >= KNN / 2 + 0.5).astype(jnp.int32)


@jax.jit
def kernel(query, queue_anchor, queue_label):
    labels_f = queue_label.astype(jnp.float32).reshape(1, K)
    out = pl.pallas_call(
        _tc_body,
        grid=(NSTEPS,),
        in_specs=[
            pl.BlockSpec((Q, DIM), lambda i: (0, 0)),
            pl.BlockSpec((KT, DIM), lambda i: (i, 0)),
            pl.BlockSpec((1, K), lambda i: (0, 0)),
        ],
        out_specs=pl.BlockSpec((Q, 1), lambda i: (0, 0)),
        out_shape=jax.ShapeDtypeStruct((Q, 1), jnp.int32),
        scratch_shapes=[
            pltpu.VMEM((Q, K), jnp.float32),
            pltpu.VMEM((Q, DIM), jnp.float32),
        ],
    )(query, queue_anchor, labels_f)
    return out.reshape(Q)


# speedup vs baseline: 3.3759x; 1.3475x over previous
import functools

import jax
import jax.numpy as jnp
from jax import lax
from jax.experimental import pallas as pl
from jax.experimental.pallas import tpu as pltpu

K = 1024
DIM = 2048
KNN = 8
Q = 32
KT = 256
NSTEPS = K // KT


def _tc_body(query_ref, anchor_ref, label_ref, out_ref, dist_ref, logq_ref):
    i = pl.program_id(0)

    @pl.when(i == 0)
    def _():
        logq_ref[...] = jnp.log(query_ref[...])

    a = anchor_ref[...]
    log_a = jnp.log(a)
    self_term = jnp.sum(a * log_a, axis=1, keepdims=True)
    cross = lax.dot_general(
        a, logq_ref[...], (((1,), (1,)), ((), ())),
        preferred_element_type=jnp.float32)
    dist_ref[:, pl.ds(i * KT, KT)] = lax.transpose(
        (self_term - cross) / DIM, (1, 0))

    @pl.when(i == NSTEPS - 1)
    def _():
        d = dist_ref[...]
        iota = lax.broadcasted_iota(jnp.int32, (Q, K), 1)
        labels = label_ref[...]
        s = jnp.zeros((Q, 1), jnp.int32)
        for _ in range(KNN):
            m = jnp.min(d, axis=1, keepdims=True)
            idx = jnp.min(jnp.where(d == m, iota, K), axis=1, keepdims=True)
            sel = iota == idx
            s = s + jnp.sum(jnp.where(sel, labels, 0), axis=1, keepdims=True)
            d = jnp.where(sel, jnp.inf, d)
        out_ref[...] = lax.transpose(
            (s >= KNN // 2 + 1).astype(jnp.int32), (1, 0))


@jax.jit
def kernel(query, queue_anchor, queue_label):
    labels_2d = queue_label.reshape(1, K)
    out = pl.pallas_call(
        _tc_body,
        grid=(NSTEPS,),
        in_specs=[
            pl.BlockSpec((Q, DIM), lambda i: (0, 0)),
            pl.BlockSpec((KT, DIM), lambda i: (i, 0)),
            pl.BlockSpec((1, K), lambda i: (0, 0)),
        ],
        out_specs=pl.BlockSpec((1, Q), lambda i: (0, 0)),
        out_shape=jax.ShapeDtypeStruct((1, Q), jnp.int32),
        scratch_shapes=[
            pltpu.VMEM((Q, K), jnp.float32),
            pltpu.VMEM((Q, DIM), jnp.float32),
        ],
    )(query, queue_anchor, labels_2d)
    return out.reshape(Q)


# speedup vs baseline: 3.6134x; 1.0703x over previous
import functools

import jax
import jax.numpy as jnp
from jax import lax
from jax.experimental import pallas as pl
from jax.experimental.pallas import tpu as pltpu

K = 1024
DIM = 2048
KNN = 8
Q = 32
KT = 512
NSTEPS = K // KT


def _tc_body(query_ref, anchor_ref, label_ref, out_ref, dist_ref, logq_ref):
    i = pl.program_id(0)

    @pl.when(i == 0)
    def _():
        logq_ref[...] = jnp.log(query_ref[...])

    a = anchor_ref[...]
    log_a = jnp.log(a)
    self_term = jnp.sum(a * log_a, axis=1, keepdims=True)
    cross = lax.dot_general(
        a, logq_ref[...], (((1,), (1,)), ((), ())),
        preferred_element_type=jnp.float32)
    dist_ref[:, pl.ds(i * KT, KT)] = lax.transpose(
        (self_term - cross) / DIM, (1, 0))

    @pl.when(i == NSTEPS - 1)
    def _():
        d = dist_ref[...]
        iota = lax.broadcasted_iota(jnp.int32, (Q, K), 1)
        labels = label_ref[...]
        s = jnp.zeros((Q, 1), jnp.int32)
        for _ in range(KNN):
            m = jnp.min(d, axis=1, keepdims=True)
            idx = jnp.min(jnp.where(d == m, iota, K), axis=1, keepdims=True)
            sel = iota == idx
            s = s + jnp.sum(jnp.where(sel, labels, 0), axis=1, keepdims=True)
            d = jnp.where(sel, jnp.inf, d)
        out_ref[...] = lax.transpose(
            (s >= KNN // 2 + 1).astype(jnp.int32), (1, 0))


@jax.jit
def kernel(query, queue_anchor, queue_label):
    labels_2d = queue_label.reshape(1, K)
    out = pl.pallas_call(
        _tc_body,
        grid=(NSTEPS,),
        in_specs=[
            pl.BlockSpec((Q, DIM), lambda i: (0, 0)),
            pl.BlockSpec((KT, DIM), lambda i: (i, 0)),
            pl.BlockSpec((1, K), lambda i: (0, 0)),
        ],
        out_specs=pl.BlockSpec((1, Q), lambda i: (0, 0)),
        out_shape=jax.ShapeDtypeStruct((1, Q), jnp.int32),
        scratch_shapes=[
            pltpu.VMEM((Q, K), jnp.float32),
            pltpu.VMEM((Q, DIM), jnp.float32),
        ],
    )(query, queue_anchor, labels_2d)
    return out.reshape(Q)
